# baseline (speedup 1.0000x reference)
import functools

import jax
import jax.numpy as jnp
from jax.experimental import pallas as pl
from jax.experimental.pallas import tpu as pltpu

B = 1024
L = 200
D = 128
NNEG = 5
PAIRS = 1 + NNEG
OUT_ROWS = B * PAIRS

BLK = 32
NSTEP = B // BLK
GR = OUT_ROWS // NSTEP

_IDX_CACHE = []


def _sample_indices():
    if not _IDX_CACHE:
        import numpy as np
        with jax.ensure_compile_time_eval(), \
             jax.default_device(jax.devices("cpu")[0]):
            nkey = jax.random.key(42)
            r = jax.random.randint(nkey, (B, NNEG), 0, B - 1)
            neg = (jnp.arange(B)[:, None] + 1 + r) % B
            z2_idx = jnp.concatenate(
                [jnp.arange(B)[:, None], neg], axis=1).reshape(-1)
            _IDX_CACHE.append(np.asarray(z2_idx, dtype=np.int32))
    return _IDX_CACHE[0]


def _fused_kernel(idx_ref, x2_ref, x1_ref, z1_ref, z2_ref, m2_ref):
    i = pl.program_id(0)
    inv_l = jnp.float32(1.0 / L)

    @pl.when(i < NSTEP)
    def _first_half():
        m2 = jnp.sum(x2_ref[...], axis=1) * inv_l
        m2_ref[pl.ds(i * BLK, BLK), :] = m2

    @pl.when(i >= NSTEP)
    def _second_half():
        g = i - NSTEP
        m1 = jnp.sum(x1_ref[...], axis=1) * inv_l
        rep = jnp.broadcast_to(m1[:, None, :], (BLK, PAIRS, D))
        z1_ref[...] = rep.reshape(BLK * PAIRS, D)
        idx = idx_ref[g]
        tbl = jax.lax.broadcasted_iota(jnp.int32, (B, GR), 0)
        oh = (tbl == idx).astype(jnp.float32)
        z2_ref[pl.ds(g * GR, GR), :] = jax.lax.dot_general(
            oh, m2_ref[...],
            dimension_numbers=(((0,), (0,)), ((), ())),
            preferred_element_type=jnp.float32,
        )


@functools.partial(jax.jit, static_argnames=())
def kernel(embeds, embeds_2, pids):
    del pids

    idx3 = jnp.asarray(_sample_indices()).reshape(NSTEP, 1, GR)

    z1_flat, z2_flat = pl.pallas_call(
        _fused_kernel,
        grid=(2 * NSTEP,),
        in_specs=[
            pl.BlockSpec((NSTEP, 1, GR), lambda i: (0, 0, 0)),
            pl.BlockSpec((BLK, L, D),
                         lambda i: (jnp.minimum(i, NSTEP - 1), 0, 0)),
            pl.BlockSpec((BLK, L, D),
                         lambda i: (jnp.maximum(i - NSTEP, 0), 0, 0)),
        ],
        out_specs=[
            pl.BlockSpec((BLK * PAIRS, D), lambda i: (jnp.maximum(i - NSTEP, 0), 0)),
            pl.BlockSpec((OUT_ROWS, D), lambda i: (0, 0)),
        ],
        out_shape=[
            jax.ShapeDtypeStruct((OUT_ROWS, D), jnp.float32),
            jax.ShapeDtypeStruct((OUT_ROWS, D), jnp.float32),
        ],
        scratch_shapes=[pltpu.VMEM((B, D), jnp.float32)],
    )(idx3, embeds_2, embeds)

    labels = jnp.tile(
        jnp.concatenate([jnp.ones((1,), jnp.float32), jnp.zeros((NNEG,), jnp.float32)]),
        B,
    )
    return (z1_flat[:, None, :], z2_flat[:, None, :], labels)


# speedup vs baseline: 1.1862x; 1.1862x over previous
import functools

import jax
import jax.numpy as jnp
from jax.experimental import pallas as pl
from jax.experimental.pallas import tpu as pltpu

B = 1024
L = 200
D = 128
NNEG = 5
PAIRS = 1 + NNEG
OUT_ROWS = B * PAIRS

BLK = 64
NSTEP = B // BLK
GR = OUT_ROWS // NSTEP

_IDX_CACHE = []


def _sample_indices():
    def build():
        nkey = jax.random.key(42)
        r = jax.random.randint(nkey, (B, NNEG), 0, B - 1)
        neg = (jnp.arange(B)[:, None] + 1 + r) % B
        return jnp.concatenate(
            [jnp.arange(B)[:, None], neg], axis=1).reshape(-1).astype(jnp.int32)

    if not _IDX_CACHE:
        import numpy as np
        try:
            with jax.ensure_compile_time_eval(), \
                 jax.default_device(jax.devices("cpu")[0]):
                _IDX_CACHE.append(np.asarray(build(), dtype=np.int32))
        except Exception:
            return build()
    return _IDX_CACHE[0]


def _fused_kernel(idx_ref, x2_ref, x1_ref, z1_ref, z2_ref, m2_ref):
    i = pl.program_id(0)
    inv_l = jnp.float32(1.0 / L)

    @pl.when(i < NSTEP)
    def _first_half():
        m2 = jnp.sum(x2_ref[...], axis=1) * inv_l
        m2_ref[pl.ds(i * BLK, BLK), :] = m2

    @pl.when(i >= NSTEP)
    def _second_half():
        g = i - NSTEP
        m1 = jnp.sum(x1_ref[...], axis=1) * inv_l
        rep = jnp.broadcast_to(m1[:, None, :], (BLK, PAIRS, D))
        z1_ref[...] = rep.reshape(BLK * PAIRS, D)
        idx = idx_ref[g]
        tbl = jax.lax.broadcasted_iota(jnp.int32, (B, GR), 0)
        oh = (tbl == idx).astype(jnp.float32)
        z2_ref[...] = jax.lax.dot_general(
            oh, m2_ref[...],
            dimension_numbers=(((0,), (0,)), ((), ())),
            preferred_element_type=jnp.float32,
        )


@functools.partial(jax.jit, static_argnames=())
def kernel(embeds, embeds_2, pids):
    del pids

    idx3 = jnp.asarray(_sample_indices()).reshape(NSTEP, 1, GR)

    z1_flat, z2_flat = pl.pallas_call(
        _fused_kernel,
        grid=(2 * NSTEP,),
        in_specs=[
            pl.BlockSpec((NSTEP, 1, GR), lambda i: (0, 0, 0)),
            pl.BlockSpec((BLK, L, D),
                         lambda i: (jnp.minimum(i, NSTEP - 1), 0, 0)),
            pl.BlockSpec((BLK, L, D),
                         lambda i: (jnp.maximum(i - NSTEP, 0), 0, 0)),
        ],
        out_specs=[
            pl.BlockSpec((BLK * PAIRS, D), lambda i: (jnp.maximum(i - NSTEP, 0), 0)),
            pl.BlockSpec((GR, D), lambda i: (jnp.maximum(i - NSTEP, 0), 0)),
        ],
        out_shape=[
            jax.ShapeDtypeStruct((OUT_ROWS, D), jnp.float32),
            jax.ShapeDtypeStruct((OUT_ROWS, D), jnp.float32),
        ],
        scratch_shapes=[pltpu.VMEM((B, D), jnp.float32)],
    )(idx3, embeds_2, embeds)

    labels = jnp.tile(
        jnp.concatenate([jnp.ones((1,), jnp.float32), jnp.zeros((NNEG,), jnp.float32)]),
        B,
    )
    return (z1_flat[:, None, :], z2_flat[:, None, :], labels)


# speedup vs baseline: 1.2070x; 1.0175x over previous
import functools

import jax
import jax.numpy as jnp
from jax.experimental import pallas as pl
from jax.experimental.pallas import tpu as pltpu

B = 1024
L = 200
D = 128
NNEG = 5
PAIRS = 1 + NNEG
OUT_ROWS = B * PAIRS

BLK = 64
NSTEP = B // BLK
GR = OUT_ROWS // NSTEP

_IDX_CACHE = []


def _sample_indices():
    def build():
        nkey = jax.random.key(42)
        r = jax.random.randint(nkey, (B, NNEG), 0, B - 1)
        neg = (jnp.arange(B)[:, None] + 1 + r) % B
        return jnp.concatenate(
            [jnp.arange(B)[:, None], neg], axis=1).reshape(-1).astype(jnp.int32)

    if not _IDX_CACHE:
        import numpy as np
        try:
            with jax.ensure_compile_time_eval(), \
                 jax.default_device(jax.devices("cpu")[0]):
                _IDX_CACHE.append(np.asarray(build(), dtype=np.int32))
        except Exception:
            return build()
    return _IDX_CACHE[0]


SUB = BLK // 2


def _fused_kernel(idx_ref, x2a_ref, x2b_ref, x1a_ref, x1b_ref,
                  z1_ref, z2_ref, m2_ref):
    i = pl.program_id(0)
    inv_l = jnp.float32(1.0 / L)

    @pl.when(i < NSTEP)
    def _first_half():
        m2_ref[pl.ds(i * BLK, SUB), :] = jnp.sum(x2a_ref[...], axis=1) * inv_l
        m2_ref[pl.ds(i * BLK + SUB, SUB), :] = jnp.sum(x2b_ref[...], axis=1) * inv_l

    @pl.when(i >= NSTEP)
    def _second_half():
        g = i - NSTEP
        m1a = jnp.sum(x1a_ref[...], axis=1) * inv_l
        m1b = jnp.sum(x1b_ref[...], axis=1) * inv_l
        repa = jnp.broadcast_to(m1a[:, None, :], (SUB, PAIRS, D))
        repb = jnp.broadcast_to(m1b[:, None, :], (SUB, PAIRS, D))
        z1_ref[pl.ds(0, SUB * PAIRS), :] = repa.reshape(SUB * PAIRS, D)
        z1_ref[pl.ds(SUB * PAIRS, SUB * PAIRS), :] = repb.reshape(SUB * PAIRS, D)
        idx = idx_ref[g]
        tbl = jax.lax.broadcasted_iota(jnp.int32, (B, GR), 0)
        oh = (tbl == idx).astype(jnp.float32)
        z2_ref[...] = jax.lax.dot_general(
            oh, m2_ref[...],
            dimension_numbers=(((0,), (0,)), ((), ())),
            preferred_element_type=jnp.float32,
        )


@functools.partial(jax.jit, static_argnames=())
def kernel(embeds, embeds_2, pids):
    del pids

    idx3 = jnp.asarray(_sample_indices()).reshape(NSTEP, 1, GR)

    z1_flat, z2_flat = pl.pallas_call(
        _fused_kernel,
        grid=(2 * NSTEP,),
        in_specs=[
            pl.BlockSpec((NSTEP, 1, GR), lambda i: (0, 0, 0)),
            pl.BlockSpec((SUB, L, D),
                         lambda i: (2 * jnp.minimum(i, NSTEP - 1), 0, 0)),
            pl.BlockSpec((SUB, L, D),
                         lambda i: (2 * jnp.minimum(i, NSTEP - 1) + 1, 0, 0)),
            pl.BlockSpec((SUB, L, D),
                         lambda i: (2 * jnp.maximum(i - NSTEP, 0), 0, 0)),
            pl.BlockSpec((SUB, L, D),
                         lambda i: (2 * jnp.maximum(i - NSTEP, 0) + 1, 0, 0)),
        ],
        out_specs=[
            pl.BlockSpec((BLK * PAIRS, D), lambda i: (jnp.maximum(i - NSTEP, 0), 0)),
            pl.BlockSpec((GR, D), lambda i: (jnp.maximum(i - NSTEP, 0), 0)),
        ],
        out_shape=[
            jax.ShapeDtypeStruct((OUT_ROWS, D), jnp.float32),
            jax.ShapeDtypeStruct((OUT_ROWS, D), jnp.float32),
        ],
        scratch_shapes=[pltpu.VMEM((B, D), jnp.float32)],
    )(idx3, embeds_2, embeds_2, embeds, embeds)

    labels = jnp.tile(
        jnp.concatenate([jnp.ones((1,), jnp.float32), jnp.zeros((NNEG,), jnp.float32)]),
        B,
    )
    return (z1_flat[:, None, :], z2_flat[:, None, :], labels)


# speedup vs baseline: 1.2078x; 1.0007x over previous
import functools

import jax
import jax.numpy as jnp
from jax.experimental import pallas as pl
from jax.experimental.pallas import tpu as pltpu

B = 1024
L = 200
D = 128
NNEG = 5
PAIRS = 1 + NNEG
OUT_ROWS = B * PAIRS

BLK = 64
NSTEP = B // BLK
GR = OUT_ROWS // NSTEP

_IDX_CACHE = []


def _sample_indices():
    def build():
        nkey = jax.random.key(42)
        r = jax.random.randint(nkey, (B, NNEG), 0, B - 1)
        neg = (jnp.arange(B)[:, None] + 1 + r) % B
        return jnp.concatenate(
            [jnp.arange(B)[:, None], neg], axis=1).reshape(-1).astype(jnp.int32)

    if not _IDX_CACHE:
        import numpy as np
        try:
            with jax.ensure_compile_time_eval(), \
                 jax.default_device(jax.devices("cpu")[0]):
                _IDX_CACHE.append(np.asarray(build(), dtype=np.int32))
        except Exception:
            return build()
    return _IDX_CACHE[0]


SUB = BLK // 4


def _fused_kernel(idx_ref, x2a_ref, x2b_ref, x2c_ref, x2d_ref,
                  x1a_ref, x1b_ref, x1c_ref, x1d_ref,
                  z1_ref, z2_ref, m2_ref):
    i = pl.program_id(0)
    inv_l = jnp.float32(1.0 / L)

    @pl.when(i < NSTEP)
    def _first_half():
        for k, ref in enumerate((x2a_ref, x2b_ref, x2c_ref, x2d_ref)):
            m2_ref[pl.ds(i * BLK + k * SUB, SUB), :] = (
                jnp.sum(ref[...], axis=1) * inv_l)

    @pl.when(i >= NSTEP)
    def _second_half():
        g = i - NSTEP
        for k, ref in enumerate((x1a_ref, x1b_ref, x1c_ref, x1d_ref)):
            m1 = jnp.sum(ref[...], axis=1) * inv_l
            rep = jnp.broadcast_to(m1[:, None, :], (SUB, PAIRS, D))
            z1_ref[pl.ds(k * SUB * PAIRS, SUB * PAIRS), :] = (
                rep.reshape(SUB * PAIRS, D))
        idx = idx_ref[g]
        tbl = jax.lax.broadcasted_iota(jnp.int32, (B, GR), 0)
        oh = (tbl == idx).astype(jnp.float32)
        z2_ref[...] = jax.lax.dot_general(
            oh, m2_ref[...],
            dimension_numbers=(((0,), (0,)), ((), ())),
            preferred_element_type=jnp.float32,
        )


@functools.partial(jax.jit, static_argnames=())
def kernel(embeds, embeds_2, pids):
    del pids

    idx3 = jnp.asarray(_sample_indices()).reshape(NSTEP, 1, GR)

    z1_flat, z2_flat = pl.pallas_call(
        _fused_kernel,
        grid=(2 * NSTEP,),
        in_specs=[
            pl.BlockSpec((NSTEP, 1, GR), lambda i: (0, 0, 0)),
        ] + [
            pl.BlockSpec((SUB, L, D),
                         functools.partial(
                             lambda k, i: (4 * jnp.minimum(i, NSTEP - 1) + k, 0, 0), k))
            for k in range(4)
        ] + [
            pl.BlockSpec((SUB, L, D),
                         functools.partial(
                             lambda k, i: (4 * jnp.maximum(i - NSTEP, 0) + k, 0, 0), k))
            for k in range(4)
        ],
        out_specs=[
            pl.BlockSpec((BLK * PAIRS, D), lambda i: (jnp.maximum(i - NSTEP, 0), 0)),
            pl.BlockSpec((GR, D), lambda i: (jnp.maximum(i - NSTEP, 0), 0)),
        ],
        out_shape=[
            jax.ShapeDtypeStruct((OUT_ROWS, D), jnp.float32),
            jax.ShapeDtypeStruct((OUT_ROWS, D), jnp.float32),
        ],
        scratch_shapes=[pltpu.VMEM((B, D), jnp.float32)],
    )(idx3, *([embeds_2] * 4), *([embeds] * 4))

    labels = jnp.tile(
        jnp.concatenate([jnp.ones((1,), jnp.float32), jnp.zeros((NNEG,), jnp.float32)]),
        B,
    )
    return (z1_flat[:, None, :], z2_flat[:, None, :], labels)
